# Initial kernel scaffold; baseline (speedup 1.0000x reference)
import jax
import jax.numpy as jnp
from jax.experimental import pallas as pl


def kernel(x, shared_gate, shared_up, shared_down, routed_gate, routed_up, routed_down, router_w, router_bias):
    raise NotImplementedError("write your pallas kernel here")


# speedup vs baseline: 1.6673x; 1.6673x over previous
import functools

import jax
import jax.numpy as jnp
from jax.experimental import pallas as pl
from jax.experimental.pallas import tpu as pltpu

S = 2048
H = 1024
I = 384
NS = 2
E = 8
TM = 512


def _moe_dense_body(x_ref, g_ref, u_ref, d_ref, rw_ref, rb_ref,
                    out_ref, usage_ref, scale_ref):
    t = pl.program_id(0)
    e = pl.program_id(1)
    x_t = x_ref[...]

    @pl.when(e == 0)
    def _routing():
        logits = jax.lax.dot_general(
            x_t, rw_ref[...], (((1,), (1,)), ((), ())),
            preferred_element_type=jnp.float32)
        logits = logits + rb_ref[...]
        sig = jax.nn.sigmoid(logits)
        col = jax.lax.broadcasted_iota(jnp.int32, (TM, E), 1)
        m1 = jnp.max(sig, axis=1, keepdims=True)
        i1 = jnp.min(jnp.where(sig == m1, col, E), axis=1, keepdims=True)
        sig2 = jnp.where(col == i1, -jnp.inf, sig)
        m2 = jnp.max(sig2, axis=1, keepdims=True)
        i2 = jnp.min(jnp.where(sig2 == m2, col, E), axis=1, keepdims=True)
        denom = m1 + m2
        w1 = m1 / denom
        w2 = m2 / denom
        ecol = jax.lax.broadcasted_iota(jnp.int32, (TM, 128), 1) - NS
        scale = (jnp.where(ecol == i1, w1, 0.0)
                 + jnp.where(ecol == i2, w2, 0.0))
        scale = jnp.where(ecol < 0, 1.0, scale)
        scale_ref[...] = scale
        ucol = jax.lax.broadcasted_iota(jnp.int32, (TM, 128), 1)
        oh = ((ucol == i1) | (ucol == i2)).astype(jnp.float32)
        contrib = jnp.sum(oh, axis=0, keepdims=True)

        @pl.when(t == 0)
        def _():
            usage_ref[...] = contrib

        @pl.when(t != 0)
        def _():
            usage_ref[...] += contrib

    g = g_ref[0]
    u = u_ref[0]
    d = d_ref[0]
    gx = jax.lax.dot_general(x_t, g, (((1,), (1,)), ((), ())),
                             preferred_element_type=jnp.float32)
    ux = jax.lax.dot_general(x_t, u, (((1,), (1,)), ((), ())),
                             preferred_element_type=jnp.float32)
    h = (gx * jax.nn.sigmoid(gx)) * ux
    contrib = jax.lax.dot_general(h, d, (((1,), (1,)), ((), ())),
                                  preferred_element_type=jnp.float32)
    scol = jax.lax.broadcasted_iota(jnp.int32, (TM, 128), 1)
    scale_col = jnp.sum(jnp.where(scol == e, scale_ref[...], 0.0),
                        axis=1, keepdims=True)
    contrib = contrib * scale_col

    @pl.when(e == 0)
    def _():
        out_ref[...] = contrib

    @pl.when(e != 0)
    def _():
        out_ref[...] += contrib


@functools.partial(jax.jit, static_argnames=())
def _moe_dense(x2d, gates, ups, downs, rw, rb):
    ntiles = S // TM
    nexp = NS + E
    out, usage = pl.pallas_call(
        _moe_dense_body,
        grid=(ntiles, nexp),
        in_specs=[
            pl.BlockSpec((TM, H), lambda t, e: (t, 0)),
            pl.BlockSpec((1, I, H), lambda t, e: (e, 0, 0)),
            pl.BlockSpec((1, I, H), lambda t, e: (e, 0, 0)),
            pl.BlockSpec((1, H, I), lambda t, e: (e, 0, 0)),
            pl.BlockSpec((E, H), lambda t, e: (0, 0)),
            pl.BlockSpec((1, E), lambda t, e: (0, 0)),
        ],
        out_specs=[
            pl.BlockSpec((TM, H), lambda t, e: (t, 0)),
            pl.BlockSpec((1, 128), lambda t, e: (0, 0)),
        ],
        out_shape=[
            jax.ShapeDtypeStruct((S, H), jnp.float32),
            jax.ShapeDtypeStruct((1, 128), jnp.float32),
        ],
        scratch_shapes=[pltpu.VMEM((TM, 128), jnp.float32)],
    )(x2d, gates, ups, downs, rw, rb)
    return out, usage


def kernel(x, shared_gate, shared_up, shared_down, routed_gate, routed_up,
           routed_down, router_w, router_bias):
    x2d = x.reshape(S, H)
    gates = jnp.concatenate([shared_gate, routed_gate], axis=0)
    ups = jnp.concatenate([shared_up, routed_up], axis=0)
    downs = jnp.concatenate([shared_down, routed_down], axis=0)
    rb = router_bias.reshape(1, E)
    out, usage = _moe_dense(x2d, gates, ups, downs, router_w, rb)
    return out.reshape(x.shape), usage[0, :E]
